# baseline (speedup 1.0000x reference)
import jax
import jax.numpy as jnp
from jax.experimental import pallas as pl
from jax.experimental.pallas import tpu as pltpu

T, E, D, F, SF = 32, 64, 1024, 512, 512
N_GROUP, TOPK_GROUP, TOP_K, RSF = 8, 4, 8, 2.5
GS = E // N_GROUP

_NEG = float("-inf")


def _roll(x, k):
    return jnp.roll(x, k, axis=1)


def _routing(logits, bias):
    scores = jax.nn.sigmoid(logits)
    swb = scores + bias
    lane = jax.lax.broadcasted_iota(jnp.int32, (T, E), 1)
    lmod = lane % GS
    gid = lane // GS

    m1, m2 = swb, jnp.full((T, E), _NEG)
    for sft in (1, 2, 4):
        ingrp = lmod >= sft
        b1 = jnp.where(ingrp, _roll(m1, sft), _roll(m1, sft - GS))
        b2 = jnp.where(ingrp, _roll(m2, sft), _roll(m2, sft - GS))
        hi = jnp.maximum(m1, b1)
        lo = jnp.minimum(m1, b1)
        m2 = jnp.maximum(lo, jnp.maximum(m2, b2))
        m1 = hi
    gscore = m1 + m2

    grank = jnp.zeros((T, E), jnp.int32)
    for k in range(1, N_GROUP):
        r = _roll(gscore, k * GS)
        rgid = (gid + (N_GROUP - k)) % N_GROUP
        beats = (r > gscore) | ((r == gscore) & (rgid < gid))
        grank += beats.astype(jnp.int32)
    group_mask = (grank < TOPK_GROUP).astype(jnp.float32)
    swb_m = swb * group_mask

    def _rank_lanes(v):
        rk = jnp.zeros((T, E), jnp.int32)
        for k in range(1, E):
            r = _roll(v, k)
            rl = jnp.where(lane >= k, lane - k, lane + (E - k))
            beats = (r > v) | ((r == v) & (rl < lane))
            rk += beats.astype(jnp.int32)
        return rk

    rank1 = _rank_lanes(swb_m)
    new_mask = (rank1 < TOP_K).astype(jnp.float32)
    s = scores * new_mask
    s = s / (jnp.sum(s, axis=1, keepdims=True) + 1e-20) * RSF

    rank2 = _rank_lanes(s)
    vals, idxs = [], []
    for p in range(TOP_K):
        sel = rank2 == p
        vals.append(jnp.sum(jnp.where(sel, s, 0.0), axis=1, keepdims=True))
        idxs.append(jnp.sum(jnp.where(sel, lane, 0), axis=1, keepdims=True))
    topk_vals = jnp.concatenate(vals, axis=1)
    topk_idx = jnp.concatenate(idxs, axis=1).astype(jnp.int32)
    return s, topk_vals, topk_idx


EPB = 2


def _moe_body(x_ref, gate_ref, bias_ref, w1_ref, w3_ref, w2_ref,
              wg_ref, wu_ref, wd_ref,
              out_ref, idx_ref, val_ref, s_ref):
    step = pl.program_id(0)
    x = x_ref[...]

    @pl.when(step == 0)
    def _prologue():
        logits = jax.lax.dot_general(
            x, gate_ref[...], (((1,), (1,)), ((), ())),
            preferred_element_type=jnp.float32)
        s, tvals, tidx = _routing(logits, bias_ref[...])
        s_ref[...] = s
        val_ref[...] = tvals
        idx_ref[...] = tidx
        g = jnp.dot(x, wg_ref[...], preferred_element_type=jnp.float32)
        u = jnp.dot(x, wu_ref[...], preferred_element_type=jnp.float32)
        sh = jnp.dot(jax.nn.silu(g) * u, wd_ref[...],
                     preferred_element_type=jnp.float32)
        out_ref[...] = sh

    xb = x.astype(jnp.bfloat16)
    lane = jax.lax.broadcasted_iota(jnp.int32, (T, E), 1)
    srow = s_ref[...]
    acts = []
    for j in range(EPB):
        e = step * EPB + j
        h1 = jnp.dot(xb, w1_ref[j].astype(jnp.bfloat16),
                     preferred_element_type=jnp.float32)
        h3 = jnp.dot(xb, w3_ref[j].astype(jnp.bfloat16),
                     preferred_element_type=jnp.float32)
        s_col = jnp.sum(jnp.where(lane == e, srow, 0.0), axis=1,
                        keepdims=True)
        acts.append(jax.nn.silu(h1) * h3 * s_col)
    act_cat = jnp.concatenate(acts, axis=1).astype(jnp.bfloat16)
    w2_cat = w2_ref[...].reshape(EPB * F, D).astype(jnp.bfloat16)
    out_ref[...] += jnp.dot(act_cat, w2_cat,
                            preferred_element_type=jnp.float32)


def kernel(hidden_states, gate_w, e_score_correction_bias, w1, w3, w2, wg, wu, wd):
    bias2d = e_score_correction_bias.reshape(1, E)
    grid = (E // EPB,)
    const = lambda e: (0, 0)
    out, idx, vals = pl.pallas_call(
        _moe_body,
        grid=grid,
        in_specs=[
            pl.BlockSpec((T, D), const),
            pl.BlockSpec((E, D), const),
            pl.BlockSpec((1, E), const),
            pl.BlockSpec((EPB, D, F), lambda e: (e, 0, 0)),
            pl.BlockSpec((EPB, D, F), lambda e: (e, 0, 0)),
            pl.BlockSpec((EPB, F, D), lambda e: (e, 0, 0)),
            pl.BlockSpec((D, SF), const),
            pl.BlockSpec((D, SF), const),
            pl.BlockSpec((SF, D), const),
        ],
        out_specs=[
            pl.BlockSpec((T, D), const),
            pl.BlockSpec((T, TOP_K), const),
            pl.BlockSpec((T, TOP_K), const),
        ],
        out_shape=[
            jax.ShapeDtypeStruct((T, D), jnp.float32),
            jax.ShapeDtypeStruct((T, TOP_K), jnp.int32),
            jax.ShapeDtypeStruct((T, TOP_K), jnp.float32),
        ],
        scratch_shapes=[pltpu.VMEM((T, E), jnp.float32)],
        compiler_params=pltpu.CompilerParams(
            dimension_semantics=("arbitrary",),
        ),
    )(hidden_states, gate_w, bias2d, w1, w3, w2, wg, wu, wd)
    return out, idx, vals
